# Initial kernel scaffold; baseline (speedup 1.0000x reference)
import jax
import jax.numpy as jnp
from jax.experimental import pallas as pl


def kernel(x, W_qkv, W_proj, b_proj):
    raise NotImplementedError("write your pallas kernel here")


# speedup vs baseline: 1.8841x; 1.8841x over previous
import functools

import jax
import jax.numpy as jnp
from jax.experimental import pallas as pl

DIM = 256
HEADS = 8
HEAD_DIM = DIM // HEADS
SCALE = HEAD_DIM ** (-0.5)
THRESH = 15.0
N_TOK = 2048
ROW_BLK = 512
N_RB = N_TOK // ROW_BLK

_NEG_INF = float("-inf")


def _prep_body(x_ref, w_ref, qkvT_ref, brow_ref, bcol_ref):
    x = x_ref[...]
    w = w_ref[...]
    qkvT = jax.lax.dot_general(
        w.astype(jnp.bfloat16), x.astype(jnp.bfloat16),
        (((1,), (1,)), ((), ())),
        preferred_element_type=jnp.float32)
    qkvT_ref[...] = qkvT.astype(jnp.bfloat16)
    nrm = jnp.sqrt(jnp.sum(x * x, axis=-1, keepdims=True))
    mx = jnp.max(nrm)
    valid = (nrm / mx * 75.0) >= THRESH
    bcol = jnp.where(valid, 0.0, _NEG_INF).astype(jnp.float32)
    bcol_ref[...] = bcol
    brow_ref[...] = bcol.reshape(1, N_TOK)


def _attn_body(q_ref, k_ref, v_ref, brow_ref, bcol_ref, o_ref):
    q = q_ref[...]
    k = k_ref[...]
    s = jax.lax.dot_general(
        q, k, (((0,), (0,)), ((), ())),
        preferred_element_type=jnp.float32)
    s = s * SCALE + bcol_ref[...] + brow_ref[...]
    m = jnp.max(s, axis=-1, keepdims=True)
    e = jnp.exp(s - m)
    r = jnp.sum(e, axis=-1, keepdims=True)
    p = (e / r).astype(jnp.bfloat16)
    o = jax.lax.dot_general(
        v_ref[...], p, (((1,), (1,)), ((), ())),
        preferred_element_type=jnp.float32)
    o_ref[...] = o.astype(jnp.bfloat16)


def _fin_body(outT_ref, wp_ref, b_ref, x_ref, o_ref):
    res = jax.lax.dot_general(
        outT_ref[...], wp_ref[...].astype(jnp.bfloat16),
        (((0,), (1,)), ((), ())),
        preferred_element_type=jnp.float32)
    o_ref[...] = res + b_ref[...] + x_ref[...]


@jax.jit
def kernel(x, W_qkv, W_proj, b_proj):
    B, T, H, W, C = x.shape
    N = T * H * W
    x_seq = x.reshape(N, C)

    qkvT, brow, bcol = pl.pallas_call(
        _prep_body,
        out_shape=(
            jax.ShapeDtypeStruct((3 * C, N), jnp.bfloat16),
            jax.ShapeDtypeStruct((1, N), jnp.float32),
            jax.ShapeDtypeStruct((N, 1), jnp.float32),
        ),
    )(x_seq, W_qkv)

    nhb = HEAD_DIM
    outT = pl.pallas_call(
        _attn_body,
        grid=(HEADS, N_RB),
        in_specs=[
            pl.BlockSpec((nhb, ROW_BLK), lambda h, rb: (h, rb)),
            pl.BlockSpec((nhb, N), lambda h, rb: (HEADS + h, 0)),
            pl.BlockSpec((nhb, N), lambda h, rb: (2 * HEADS + h, 0)),
            pl.BlockSpec((1, N), lambda h, rb: (0, 0)),
            pl.BlockSpec((ROW_BLK, 1), lambda h, rb: (rb, 0)),
        ],
        out_specs=pl.BlockSpec((nhb, ROW_BLK), lambda h, rb: (h, rb)),
        out_shape=jax.ShapeDtypeStruct((C, N), jnp.bfloat16),
    )(qkvT, qkvT, qkvT, brow, bcol)

    out = pl.pallas_call(
        _fin_body,
        out_shape=jax.ShapeDtypeStruct((N, C), jnp.float32),
    )(outT, W_proj, b_proj.reshape(1, C), x_seq)

    return out.reshape(B, T, H, W, C)


# speedup vs baseline: 2.3556x; 1.2503x over previous
import math

import jax
import jax.numpy as jnp
from jax.experimental import pallas as pl

DIM = 256
HEADS = 8
HEAD_DIM = DIM // HEADS
SCALE = HEAD_DIM ** (-0.5)
THRESH = 15.0
N_TOK = 2048
ROW_BLK = 512
N_RB = N_TOK // ROW_BLK
LOG2E = math.log2(math.e)

_NAN = float("nan")


def _prep_body(x_ref, w_ref, qkvT_ref, v01_ref, nan_ref):
    x = x_ref[...]
    w = w_ref[...]
    qkvT = jax.lax.dot_general(
        w.astype(jnp.bfloat16), x.astype(jnp.bfloat16),
        (((1,), (1,)), ((), ())),
        preferred_element_type=jnp.float32)
    qkvT_ref[:DIM, :] = (qkvT[:DIM, :] * (SCALE * LOG2E)).astype(jnp.bfloat16)
    qkvT_ref[DIM:, :] = qkvT[DIM:, :].astype(jnp.bfloat16)
    nrm = jnp.sqrt(jnp.sum(x * x, axis=-1, keepdims=True))
    mx = jnp.max(nrm)
    valid = (nrm / mx * 75.0) >= THRESH
    nan_ref[...] = jnp.where(valid, 0.0, _NAN).astype(jnp.float32)
    v01 = jnp.where(valid, 1.0, 0.0).astype(jnp.float32)
    v01_ref[...] = v01.reshape(1, N_TOK).astype(jnp.bfloat16)


def _attn_body(q_ref, k_ref, v_ref, v01_ref, o_ref):
    q = q_ref[...]
    k = k_ref[...]
    s = jax.lax.dot_general(
        q, k, (((0,), (0,)), ((), ())),
        preferred_element_type=jnp.float32)
    m = jnp.max(s, axis=-1, keepdims=True)
    e = jnp.exp2(s - m).astype(jnp.bfloat16) * v01_ref[...]
    vcat = jnp.concatenate(
        [v_ref[...], jnp.ones((8, N_TOK), jnp.bfloat16)], axis=0)
    oa = jax.lax.dot_general(
        vcat, e, (((1,), (1,)), ((), ())),
        preferred_element_type=jnp.float32)
    o = oa[:HEAD_DIM, :] / oa[HEAD_DIM:HEAD_DIM + 1, :]
    o_ref[...] = o.astype(jnp.bfloat16)


def _fin_body(outT_ref, wp_ref, b_ref, x_ref, nan_ref, o_ref):
    res = jax.lax.dot_general(
        outT_ref[...], wp_ref[...].astype(jnp.bfloat16),
        (((0,), (1,)), ((), ())),
        preferred_element_type=jnp.float32)
    o_ref[...] = res + b_ref[...] + x_ref[...] + nan_ref[...]


@jax.jit
def kernel(x, W_qkv, W_proj, b_proj):
    B, T, H, W, C = x.shape
    N = T * H * W
    x_seq = x.reshape(N, C)

    qkvT, v01, nanv = pl.pallas_call(
        _prep_body,
        out_shape=(
            jax.ShapeDtypeStruct((3 * C, N), jnp.bfloat16),
            jax.ShapeDtypeStruct((1, N), jnp.bfloat16),
            jax.ShapeDtypeStruct((N, 1), jnp.float32),
        ),
    )(x_seq, W_qkv)

    nhb = HEAD_DIM
    outT = pl.pallas_call(
        _attn_body,
        grid=(HEADS, N_RB),
        in_specs=[
            pl.BlockSpec((nhb, ROW_BLK), lambda h, rb: (h, rb)),
            pl.BlockSpec((nhb, N), lambda h, rb: (HEADS + h, 0)),
            pl.BlockSpec((nhb, N), lambda h, rb: (2 * HEADS + h, 0)),
            pl.BlockSpec((1, N), lambda h, rb: (0, 0)),
        ],
        out_specs=pl.BlockSpec((nhb, ROW_BLK), lambda h, rb: (h, rb)),
        out_shape=jax.ShapeDtypeStruct((C, N), jnp.bfloat16),
    )(qkvT, qkvT, qkvT, v01)

    out = pl.pallas_call(
        _fin_body,
        out_shape=jax.ShapeDtypeStruct((N, C), jnp.float32),
    )(outT, W_proj, b_proj.reshape(1, C), x_seq, nanv)

    return out.reshape(B, T, H, W, C)


# speedup vs baseline: 2.3658x; 1.0043x over previous
import math

import jax
import jax.numpy as jnp
from jax.experimental import pallas as pl

DIM = 256
HEADS = 8
HEAD_DIM = DIM // HEADS
SCALE = HEAD_DIM ** (-0.5)
THRESH = 15.0
N_TOK = 2048
ROW_BLK = 512
N_RB = N_TOK // ROW_BLK
LOG2E = math.log2(math.e)

_NAN = float("nan")


def _prep_body(x_ref, w_ref, qkvT_ref, v01_ref, nan_ref):
    x = x_ref[...]
    w = w_ref[...]
    qkvT = jax.lax.dot_general(
        w.astype(jnp.bfloat16), x.astype(jnp.bfloat16),
        (((1,), (1,)), ((), ())),
        preferred_element_type=jnp.float32)
    nrm = jnp.sqrt(jnp.sum(x * x, axis=-1, keepdims=True))
    mx = jnp.max(nrm)
    valid = (nrm / mx * 75.0) >= THRESH
    nan_ref[...] = jnp.where(valid, 0.0, _NAN).astype(jnp.float32)
    v01row = jnp.where(valid, 1.0, 0.0).reshape(1, N_TOK)
    v01_ref[...] = v01row.astype(jnp.bfloat16)
    qkvT_ref[:DIM, :] = (qkvT[:DIM, :] * (SCALE * LOG2E)).astype(jnp.bfloat16)
    qkvT_ref[DIM:2 * DIM, :] = qkvT[DIM:2 * DIM, :].astype(jnp.bfloat16)
    qkvT_ref[2 * DIM:, :] = (qkvT[2 * DIM:, :] * v01row).astype(jnp.bfloat16)


def _attn_body(q_ref, k_ref, v_ref, v01_ref, o_ref):
    q = q_ref[...]
    k = k_ref[...]
    s = jax.lax.dot_general(
        q, k, (((0,), (0,)), ((), ())),
        preferred_element_type=jnp.float32).astype(jnp.bfloat16)
    m = jnp.max(s, axis=-1, keepdims=True)
    e = jnp.exp2(s - m)
    vcat = jnp.concatenate(
        [v_ref[...], jnp.broadcast_to(v01_ref[...], (8, N_TOK))], axis=0)
    oa = jax.lax.dot_general(
        vcat, e, (((1,), (1,)), ((), ())),
        preferred_element_type=jnp.float32)
    o = oa[:HEAD_DIM, :] / oa[HEAD_DIM:HEAD_DIM + 1, :]
    o_ref[...] = o.astype(jnp.bfloat16)


def _fin_body(outT_ref, wp_ref, b_ref, x_ref, nan_ref, o_ref):
    res = jax.lax.dot_general(
        outT_ref[...], wp_ref[...].astype(jnp.bfloat16),
        (((0,), (1,)), ((), ())),
        preferred_element_type=jnp.float32)
    o_ref[...] = res + b_ref[...] + x_ref[...] + nan_ref[...]


@jax.jit
def kernel(x, W_qkv, W_proj, b_proj):
    B, T, H, W, C = x.shape
    N = T * H * W
    x_seq = x.reshape(N, C)

    qkvT, v01, nanv = pl.pallas_call(
        _prep_body,
        out_shape=(
            jax.ShapeDtypeStruct((3 * C, N), jnp.bfloat16),
            jax.ShapeDtypeStruct((1, N), jnp.bfloat16),
            jax.ShapeDtypeStruct((N, 1), jnp.float32),
        ),
    )(x_seq, W_qkv)

    nhb = HEAD_DIM
    outT = pl.pallas_call(
        _attn_body,
        grid=(HEADS, N_RB),
        in_specs=[
            pl.BlockSpec((nhb, ROW_BLK), lambda h, rb: (h, rb)),
            pl.BlockSpec((nhb, N), lambda h, rb: (HEADS + h, 0)),
            pl.BlockSpec((nhb, N), lambda h, rb: (2 * HEADS + h, 0)),
            pl.BlockSpec((1, N), lambda h, rb: (0, 0)),
        ],
        out_specs=pl.BlockSpec((nhb, ROW_BLK), lambda h, rb: (h, rb)),
        out_shape=jax.ShapeDtypeStruct((C, N), jnp.bfloat16),
    )(qkvT, qkvT, qkvT, v01)

    out = pl.pallas_call(
        _fin_body,
        out_shape=jax.ShapeDtypeStruct((N, C), jnp.float32),
    )(outT, W_proj, b_proj.reshape(1, C), x_seq, nanv)

    return out.reshape(B, T, H, W, C)


# speedup vs baseline: 2.8718x; 1.2139x over previous
import math

import jax
import jax.numpy as jnp
from jax.experimental import pallas as pl

DIM = 256
HEADS = 8
HEAD_DIM = DIM // HEADS
SCALE = HEAD_DIM ** (-0.5)
THRESH = 15.0
N_TOK = 2048
ROW_BLK = 2048
N_RB = N_TOK // ROW_BLK
LOG2E = math.log2(math.e)

_NAN = float("nan")


def _prep_body(x_ref, w_ref, qkvT_ref, v01_ref, nan_ref):
    x = x_ref[...]
    w = w_ref[...]
    qkvT = jax.lax.dot_general(
        w.astype(jnp.bfloat16), x.astype(jnp.bfloat16),
        (((1,), (1,)), ((), ())),
        preferred_element_type=jnp.float32)
    nrm = jnp.sqrt(jnp.sum(x * x, axis=-1, keepdims=True))
    mx = jnp.max(nrm)
    valid = (nrm / mx * 75.0) >= THRESH
    nan_ref[...] = jnp.where(valid, 0.0, _NAN).astype(jnp.float32)
    v01row = jnp.where(valid, 1.0, 0.0).reshape(1, N_TOK)
    v01_ref[...] = v01row.astype(jnp.bfloat16)
    qkvT_ref[:DIM, :] = (qkvT[:DIM, :] * (SCALE * LOG2E)).astype(jnp.bfloat16)
    qkvT_ref[DIM:2 * DIM, :] = qkvT[DIM:2 * DIM, :].astype(jnp.bfloat16)
    qkvT_ref[2 * DIM:, :] = (qkvT[2 * DIM:, :] * v01row).astype(jnp.bfloat16)


def _attn_body(q_ref, k_ref, v_ref, v01_ref, o_ref):
    q = q_ref[...]
    k = k_ref[...]
    s = jax.lax.dot_general(
        q, k, (((0,), (0,)), ((), ())),
        preferred_element_type=jnp.float32).astype(jnp.bfloat16)
    m = jnp.max(s, axis=-1, keepdims=True)
    e = jnp.exp2(s - m)
    vcat = jnp.concatenate(
        [v_ref[...], jnp.broadcast_to(v01_ref[...], (8, N_TOK))], axis=0)
    oa = jax.lax.dot_general(
        vcat, e, (((1,), (1,)), ((), ())),
        preferred_element_type=jnp.float32)
    o = oa[:HEAD_DIM, :] / oa[HEAD_DIM:HEAD_DIM + 1, :]
    o_ref[...] = o.astype(jnp.bfloat16)


def _fin_body(outT_ref, wp_ref, b_ref, x_ref, nan_ref, o_ref):
    res = jax.lax.dot_general(
        outT_ref[...], wp_ref[...].astype(jnp.bfloat16),
        (((0,), (1,)), ((), ())),
        preferred_element_type=jnp.float32)
    o_ref[...] = res + b_ref[...] + x_ref[...] + nan_ref[...]


@jax.jit
def kernel(x, W_qkv, W_proj, b_proj):
    B, T, H, W, C = x.shape
    N = T * H * W
    x_seq = x.reshape(N, C)

    qkvT, v01, nanv = pl.pallas_call(
        _prep_body,
        out_shape=(
            jax.ShapeDtypeStruct((3 * C, N), jnp.bfloat16),
            jax.ShapeDtypeStruct((1, N), jnp.bfloat16),
            jax.ShapeDtypeStruct((N, 1), jnp.float32),
        ),
    )(x_seq, W_qkv)

    nhb = HEAD_DIM
    outT = pl.pallas_call(
        _attn_body,
        grid=(HEADS, N_RB),
        in_specs=[
            pl.BlockSpec((nhb, ROW_BLK), lambda h, rb: (h, rb)),
            pl.BlockSpec((nhb, N), lambda h, rb: (HEADS + h, 0)),
            pl.BlockSpec((nhb, N), lambda h, rb: (2 * HEADS + h, 0)),
            pl.BlockSpec((1, N), lambda h, rb: (0, 0)),
        ],
        out_specs=pl.BlockSpec((nhb, ROW_BLK), lambda h, rb: (h, rb)),
        out_shape=jax.ShapeDtypeStruct((C, N), jnp.bfloat16),
    )(qkvT, qkvT, qkvT, v01)

    out = pl.pallas_call(
        _fin_body,
        out_shape=jax.ShapeDtypeStruct((N, C), jnp.float32),
    )(outT, W_proj, b_proj.reshape(1, C), x_seq, nanv)

    return out.reshape(B, T, H, W, C)
